# baseline (speedup 1.0000x reference)
import functools
import math

import jax
import jax.numpy as jnp
import numpy as np
from jax.experimental import pallas as pl

_z = np.random.RandomState(0).randn(1000000)
_SSP_C = float(1.0 / np.sqrt(np.mean((np.logaddexp(0.0, _z) - np.log(2.0)) ** 2)))
_LOG2 = float(np.log(2.0))


def _ssp(x):
    return (jax.nn.softplus(x) - _LOG2) * _SSP_C


def _dot(a, b):
    return jax.lax.dot_general(a, b, (((1,), (0,)), ((), ())),
                               preferred_element_type=jnp.float32)


def _node_body(x_ref, Wpre, bpre, G1, g1b, G2, g2b, Wnode, bnode, l01t, l01b,
               a_ref, b_ref, xl_ref):
    x = x_ref[...]
    D = x.shape[1]
    pre = _dot(x, Wpre[...]) * (1.0 / math.sqrt(D)) + bpre[...]
    s = 1.0 / math.sqrt(2 * D)
    a_ref[...] = _dot(pre, l01t[...]) * s
    b_ref[...] = _dot(pre, l01b[...]) * s
    h = jax.nn.silu(_dot(x, G1[...]) + g1b[...])
    xg = _dot(h, G2[...]) + g2b[...]
    xl_ref[...] = _dot(xg, Wnode[...]) * (1.0 / math.sqrt(D)) + bnode[...]


def _edge_body(ea_ref, vraw_ref, xlsh_ref, fc1, fc2, l02, ef_ref):
    H = fc2.shape[0]
    t = _ssp(_dot(ea_ref[...], fc1[...]) * (1.0 / math.sqrt(fc1.shape[0])))
    p = _dot(t, fc2[...]) * (1.0 / math.sqrt(H))
    u = _ssp(vraw_ref[...])
    q = _dot(u, l02[...]) * (1.0 / math.sqrt(H))
    ef_ref[...] = xlsh_ref[...] * p * q


def _out_body(acc_ref, xl_ref, Wout, bout, o_ref):
    D = Wout.shape[0]
    o_ref[...] = _dot(acc_ref[...] + xl_ref[...], Wout[...]) * (1.0 / math.sqrt(D)) + bout[...]


def _full(shape):
    return pl.BlockSpec(shape, lambda i: (0,) * len(shape))


def kernel(x, edge_index, edge_sh, edge_attr, W_pre, b_pre, W_node, b_node,
           G1, g1b, G2, g2b, fc1, fc2, l01, l02, W_out, b_out):
    N, D = x.shape
    E = edge_index.shape[1]
    H = fc2.shape[0]
    dst = edge_index[0]
    src = edge_index[1]
    l01t = l01[:D]
    l01b = l01[D:]
    b_pre2 = b_pre.reshape(1, D)
    b_node2 = b_node.reshape(1, D)
    g1b2 = g1b.reshape(1, D)
    g2b2 = g2b.reshape(1, D)
    b_out2 = b_out.reshape(1, D)

    BN = 2000
    n_blocks = N // BN
    a_nd, b_nd, xl = pl.pallas_call(
        _node_body,
        grid=(n_blocks,),
        in_specs=[
            pl.BlockSpec((BN, D), lambda i: (i, 0)),
            _full((D, D)), _full((1, D)),
            _full((D, D)), _full((1, D)),
            _full((D, D)), _full((1, D)),
            _full((D, D)), _full((1, D)),
            _full((D, H)), _full((D, H)),
        ],
        out_specs=[
            pl.BlockSpec((BN, H), lambda i: (i, 0)),
            pl.BlockSpec((BN, H), lambda i: (i, 0)),
            pl.BlockSpec((BN, D), lambda i: (i, 0)),
        ],
        out_shape=[
            jax.ShapeDtypeStruct((N, H), jnp.float32),
            jax.ShapeDtypeStruct((N, H), jnp.float32),
            jax.ShapeDtypeStruct((N, D), jnp.float32),
        ],
    )(x, W_pre, b_pre2, G1, g1b2, G2, g2b2, W_node, b_node2, l01t, l01b)

    vraw = a_nd[dst] + b_nd[src]
    xlsh = xl[src] * edge_sh

    BE = 8000
    e_blocks = E // BE
    ef = pl.pallas_call(
        _edge_body,
        grid=(e_blocks,),
        in_specs=[
            pl.BlockSpec((BE, edge_attr.shape[1]), lambda i: (i, 0)),
            pl.BlockSpec((BE, H), lambda i: (i, 0)),
            pl.BlockSpec((BE, D), lambda i: (i, 0)),
            _full((edge_attr.shape[1], H)), _full((H, D)), _full((H, D)),
        ],
        out_specs=pl.BlockSpec((BE, D), lambda i: (i, 0)),
        out_shape=jax.ShapeDtypeStruct((E, D), jnp.float32),
    )(edge_attr, vraw, xlsh, fc1, fc2, l02)

    acc = jnp.zeros((N, D), jnp.float32).at[dst].add(ef)

    out = pl.pallas_call(
        _out_body,
        grid=(n_blocks,),
        in_specs=[
            pl.BlockSpec((BN, D), lambda i: (i, 0)),
            pl.BlockSpec((BN, D), lambda i: (i, 0)),
            _full((D, D)), _full((1, D)),
        ],
        out_specs=pl.BlockSpec((BN, D), lambda i: (i, 0)),
        out_shape=jax.ShapeDtypeStruct((N, D), jnp.float32),
    )(acc, xl, W_out, b_out2)
    return out


# speedup vs baseline: 1.6522x; 1.6522x over previous
import functools
import math

import jax
import jax.numpy as jnp
import numpy as np
from jax import lax
from jax.experimental import pallas as pl
from jax.experimental.pallas import tpu as pltpu
from jax.experimental.pallas import tpu_sc as plsc

_z = np.random.RandomState(0).randn(1000000)
_SSP_C = float(1.0 / np.sqrt(np.mean((np.logaddexp(0.0, _z) - np.log(2.0)) ** 2)))
_LOG2 = float(np.log(2.0))

_NC = 2
_NS = 16
_NW = _NC * _NS


def _ssp(x):
    return (jax.nn.softplus(x) - _LOG2) * _SSP_C


def _dot(a, b):
    return jax.lax.dot_general(a, b, (((1,), (0,)), ((), ())),
                               preferred_element_type=jnp.float32)


def _node_body(x_ref, Wpre, bpre, G1, g1b, G2, g2b, Wnode, bnode, l01t, l01b,
               P_ref, xl_ref):
    x = x_ref[...]
    BN, D = x.shape
    H = l01t.shape[1]
    pre = _dot(x, Wpre[...]) * (1.0 / math.sqrt(D)) + bpre[...]
    s = 1.0 / math.sqrt(2 * D)
    a = _dot(pre, l01t[...]) * s
    b = _dot(pre, l01b[...]) * s
    P_ref[...] = jnp.concatenate(
        [a, b, jnp.zeros((BN, D - 2 * H), jnp.float32)], axis=1)
    h = jax.nn.silu(_dot(x, G1[...]) + g1b[...])
    xg = _dot(h, G2[...]) + g2b[...]
    xl_ref[...] = _dot(xg, Wnode[...]) * (1.0 / math.sqrt(D)) + bnode[...]


def _edge_body(ea_ref, vL_ref, vR_ref, sh_ref, fc1, fc2, l02, m_ref):
    H = fc2.shape[0]
    EA = fc1.shape[0]
    sf = 1.0 / math.sqrt(H)
    ea = ea_ref[...]
    vL = vL_ref[...]
    vR = vR_ref[...]
    sh = sh_ref[...]
    for g in range(8):
        ea_g = ea[:, g * EA:(g + 1) * EA]
        t = _ssp(_dot(ea_g, fc1[...]) * (1.0 / math.sqrt(EA)))
        p = _dot(t, fc2[...]) * sf
        v = vL if g < 4 else vR
        u = v[:, (g % 4) * H:((g % 4) + 1) * H]
        q = _dot(_ssp(u), l02[...]) * sf
        m_ref[:, g, :] = sh[:, g:g + 1] * p * q


def _out_body(p0_ref, p1_ref, xl_ref, Wout, bout, o_ref):
    D = Wout.shape[0]
    acc = p0_ref[0] + p1_ref[0] + xl_ref[...]
    o_ref[...] = _dot(acc, Wout[...]) * (1.0 / math.sqrt(D)) + bout[...]


def _full(shape):
    return pl.BlockSpec(shape, lambda i: (0,) * len(shape))


def _make_gather_kernel(D, H, E_pad):
    EPW = E_pad // _NW
    CH = 256
    OUTER = EPW // 1024
    mesh = plsc.VectorSubcoreMesh(core_axis_name="c", subcore_axis_name="s")

    @functools.partial(
        pl.kernel,
        out_type=[
            jax.ShapeDtypeStruct((E_pad // 8, D), jnp.float32),
            jax.ShapeDtypeStruct((E_pad // 8, D), jnp.float32),
        ],
        scratch_types=[
            pltpu.VMEM((8, 128), jnp.int32),
            pltpu.VMEM((8, 128), jnp.int32),
            pltpu.VMEM((CH, D), jnp.float32),
            pltpu.VMEM((CH, D), jnp.float32),
            pltpu.VMEM((CH // 8, D), jnp.float32),
            pltpu.VMEM((CH // 8, D), jnp.float32),
            pltpu.SemaphoreType.DMA,
        ],
        mesh=mesh,
    )
    def g_kernel(P_hbm, dst2_hbm, src2_hbm, vL_hbm, vR_hbm,
                 idx_d, idx_s, gd_buf, gs_buf, sumL, sumR, sem):
        cid = lax.axis_index("c")
        sid = lax.axis_index("s")
        wid = sid * _NC + cid

        def outer(i, _):
            r0 = wid * (EPW // 128) + i * 8
            pltpu.sync_copy(dst2_hbm.at[pl.ds(r0, 8)], idx_d)
            pltpu.sync_copy(src2_hbm.at[pl.ds(r0, 8)], idx_s)

            def sub(s_, _2):
                copies = []
                for bk in range(CH // 128):
                    copies.append(pltpu.async_copy(
                        P_hbm.at[idx_d.at[s_ * 2 + bk]],
                        gd_buf.at[pl.ds(bk * 128, 128)], sem))
                    copies.append(pltpu.async_copy(
                        P_hbm.at[idx_s.at[s_ * 2 + bk]],
                        gs_buf.at[pl.ds(bk * 128, 128)], sem))
                for cp in copies:
                    cp.wait()

                def addrow(r, _3):
                    for j in range(8):
                        for k in range(H // 16):
                            val = (gd_buf[r * 8 + j, pl.ds(k * 16, 16)]
                                   + gs_buf[r * 8 + j, pl.ds(H + k * 16, 16)])
                            if j < 4:
                                sumL[r, pl.ds(j * H + k * 16, 16)] = val
                            else:
                                sumR[r, pl.ds((j - 4) * H + k * 16, 16)] = val
                    return _3
                lax.fori_loop(0, CH // 8, addrow, 0)

                v0 = wid * (EPW // 8) + i * 128 + s_ * (CH // 8)
                pltpu.sync_copy(sumL, vL_hbm.at[pl.ds(v0, CH // 8)])
                pltpu.sync_copy(sumR, vR_hbm.at[pl.ds(v0, CH // 8)])
                return _2

            lax.fori_loop(0, 4, sub, 0)
            return _

        lax.fori_loop(0, OUTER, outer, 0)

    return g_kernel


def _make_scatter_kernel(N, D, E_pad):
    EPW = E_pad // _NW
    CH = 128
    OUTER = EPW // 1024
    N_pad = -(-N // (_NS * 128)) * (_NS * 128)
    RPT = N_pad // _NS
    mesh = plsc.VectorSubcoreMesh(core_axis_name="c", subcore_axis_name="s")

    @functools.partial(
        pl.kernel,
        out_type=jax.ShapeDtypeStruct((_NC, N_pad, D), jnp.float32),
        scratch_types=[
            pltpu.VMEM((8, 128), jnp.int32),
            pltpu.VMEM((8, 128), jnp.int32),
            pltpu.VMEM((CH, D), jnp.float32),
            pltpu.VMEM((CH, D), jnp.float32),
            pltpu.VMEM_SHARED((N_pad, D), jnp.float32),
            pltpu.SemaphoreType.DMA,
        ],
        mesh=mesh,
    )
    def s_kernel(xl_hbm, m_hbm, dst2_hbm, src2_hbm, part_hbm,
                 idx_d, idx_s, xl_buf, m_buf, acc, sem):
        cid = lax.axis_index("c")
        sid = lax.axis_index("s")
        wid = sid * _NC + cid

        def zrow(r, _):
            for k in range(D // 16):
                m_buf[r, pl.ds(k * 16, 16)] = jnp.zeros((16,), jnp.float32)
            return _
        lax.fori_loop(0, CH, zrow, 0)
        for j in range(RPT // CH):
            pltpu.sync_copy(m_buf, acc.at[pl.ds(sid * RPT + j * CH, CH)])
        rem = RPT % CH
        if rem:
            pltpu.sync_copy(m_buf.at[pl.ds(0, rem)],
                            acc.at[pl.ds(sid * RPT + (RPT // CH) * CH, rem)])
        plsc.subcore_barrier()

        def outer(i, _):
            r0 = wid * (EPW // 128) + i * 8
            pltpu.sync_copy(dst2_hbm.at[pl.ds(r0, 8)], idx_d)
            pltpu.sync_copy(src2_hbm.at[pl.ds(r0, 8)], idx_s)

            def sub(s_, _2):
                e0 = wid * EPW + i * 1024 + s_ * CH
                copies = [
                    pltpu.async_copy(m_hbm.at[pl.ds(e0, CH)], m_buf, sem),
                    pltpu.async_copy(xl_hbm.at[idx_s.at[s_]], xl_buf, sem),
                ]
                for cp in copies:
                    cp.wait()

                def mrow(r, _3):
                    for k in range(D // 16):
                        sl = pl.ds(k * 16, 16)
                        m_buf[r, sl] = m_buf[r, sl] * xl_buf[r, sl]
                    return _3
                lax.fori_loop(0, CH, mrow, 0)

                pltpu.sync_copy(m_buf, acc.at[idx_d.at[s_]], add=True)
                return _2

            lax.fori_loop(0, 1024 // CH, sub, 0)
            return _

        lax.fori_loop(0, OUTER, outer, 0)
        plsc.subcore_barrier()
        pltpu.sync_copy(acc.at[pl.ds(sid * RPT, RPT)],
                        part_hbm.at[cid, pl.ds(sid * RPT, RPT)])

    return s_kernel, N_pad


def kernel(x, edge_index, edge_sh, edge_attr, W_pre, b_pre, W_node, b_node,
           G1, g1b, G2, g2b, fc1, fc2, l01, l02, W_out, b_out):
    N, D = x.shape
    E = edge_index.shape[1]
    EA = edge_attr.shape[1]
    H = fc2.shape[0]
    l01t = l01[:D]
    l01b = l01[D:]
    b_pre2 = b_pre.reshape(1, D)
    b_node2 = b_node.reshape(1, D)
    g1b2 = g1b.reshape(1, D)
    g2b2 = g2b.reshape(1, D)
    b_out2 = b_out.reshape(1, D)

    BN = 2000
    n_blocks = N // BN
    P, xl = pl.pallas_call(
        _node_body,
        grid=(n_blocks,),
        in_specs=[
            pl.BlockSpec((BN, D), lambda i: (i, 0)),
            _full((D, D)), _full((1, D)),
            _full((D, D)), _full((1, D)),
            _full((D, D)), _full((1, D)),
            _full((D, D)), _full((1, D)),
            _full((D, H)), _full((D, H)),
        ],
        out_specs=[
            pl.BlockSpec((BN, D), lambda i: (i, 0)),
            pl.BlockSpec((BN, D), lambda i: (i, 0)),
        ],
        out_shape=[
            jax.ShapeDtypeStruct((N, D), jnp.float32),
            jax.ShapeDtypeStruct((N, D), jnp.float32),
        ],
    )(x, W_pre, b_pre2, G1, g1b2, G2, g2b2, W_node, b_node2, l01t, l01b)

    EPW = -(-E // (_NW * 1024)) * 1024
    E_pad = EPW * _NW
    pad = E_pad - E
    dst = jnp.pad(edge_index[0], (0, pad))
    src = jnp.pad(edge_index[1], (0, pad))
    ea_pad = jnp.pad(edge_attr, ((0, pad), (0, 0)))
    sh_pad = jnp.pad(edge_sh, ((0, pad), (0, 0)))
    dst2 = dst.reshape(E_pad // 128, 128)
    src2 = src.reshape(E_pad // 128, 128)
    ea8 = ea_pad.reshape(E_pad // 8, 8 * EA)
    sh8 = sh_pad.reshape(E_pad // 8, 8)

    vL, vR = _make_gather_kernel(D, H, E_pad)(P, dst2, src2)

    BE = 8192
    B8 = BE // 8
    e_blocks = E_pad // BE
    m3 = pl.pallas_call(
        _edge_body,
        grid=(e_blocks,),
        in_specs=[
            pl.BlockSpec((B8, 8 * EA), lambda i: (i, 0)),
            pl.BlockSpec((B8, D), lambda i: (i, 0)),
            pl.BlockSpec((B8, D), lambda i: (i, 0)),
            pl.BlockSpec((B8, 8), lambda i: (i, 0)),
            _full((EA, H)), _full((H, D)), _full((H, D)),
        ],
        out_specs=pl.BlockSpec((B8, 8, D), lambda i: (i, 0, 0)),
        out_shape=jax.ShapeDtypeStruct((E_pad // 8, 8, D), jnp.float32),
    )(ea8, vL, vR, sh8, fc1, fc2, l02)
    m = m3.reshape(E_pad, D)

    s_kernel, N_pad = _make_scatter_kernel(N, D, E_pad)
    part = s_kernel(xl, m, dst2, src2)

    out = pl.pallas_call(
        _out_body,
        grid=(n_blocks,),
        in_specs=[
            pl.BlockSpec((1, BN, D), lambda i: (0, i, 0)),
            pl.BlockSpec((1, BN, D), lambda i: (1, i, 0)),
            pl.BlockSpec((BN, D), lambda i: (i, 0)),
            _full((D, D)), _full((1, D)),
        ],
        out_specs=pl.BlockSpec((BN, D), lambda i: (i, 0)),
        out_shape=jax.ShapeDtypeStruct((N, D), jnp.float32),
    )(part, part, xl, W_out, b_out2)
    return out


# speedup vs baseline: 1.7245x; 1.0438x over previous
import functools
import math

import jax
import jax.numpy as jnp
import numpy as np
from jax import lax
from jax.experimental import pallas as pl
from jax.experimental.pallas import tpu as pltpu
from jax.experimental.pallas import tpu_sc as plsc

_z = np.random.RandomState(0).randn(1000000)
_SSP_C = float(1.0 / np.sqrt(np.mean((np.logaddexp(0.0, _z) - np.log(2.0)) ** 2)))
_LOG2 = float(np.log(2.0))

_NC = 2
_NS = 16
_NW = _NC * _NS


def _ssp(x):
    return (jax.nn.softplus(x) - _LOG2) * _SSP_C


def _dot(a, b):
    return jax.lax.dot_general(a, b, (((1,), (0,)), ((), ())),
                               preferred_element_type=jnp.float32)


def _node_body(x_ref, Wpre, bpre, G1, g1b, G2, g2b, Wnode, bnode, l01t, l01b,
               P_ref, xl_ref):
    x = x_ref[...]
    BN, D = x.shape
    H = l01t.shape[1]
    pre = _dot(x, Wpre[...]) * (1.0 / math.sqrt(D)) + bpre[...]
    s = 1.0 / math.sqrt(2 * D)
    a = _dot(pre, l01t[...]) * s
    b = _dot(pre, l01b[...]) * s
    P_ref[...] = jnp.concatenate(
        [a, b, jnp.zeros((BN, D - 2 * H), jnp.float32)], axis=1)
    h = jax.nn.silu(_dot(x, G1[...]) + g1b[...])
    xg = _dot(h, G2[...]) + g2b[...]
    xl_ref[...] = _dot(xg, Wnode[...]) * (1.0 / math.sqrt(D)) + bnode[...]


def _edge_body(ea_ref, vL_ref, vR_ref, sh_ref, fc1, fc2, l02, m_ref):
    H = fc2.shape[0]
    EA = fc1.shape[0]
    sf = 1.0 / math.sqrt(H)
    ea = ea_ref[...]
    vL = vL_ref[...]
    vR = vR_ref[...]
    sh = sh_ref[...]
    for g in range(8):
        ea_g = ea[:, g * EA:(g + 1) * EA]
        t = _ssp(_dot(ea_g, fc1[...]) * (1.0 / math.sqrt(EA)))
        p = _dot(t, fc2[...]) * sf
        v = vL if g < 4 else vR
        u = v[:, (g % 4) * H:((g % 4) + 1) * H]
        q = _dot(_ssp(u), l02[...]) * sf
        m_ref[:, g, :] = sh[:, g:g + 1] * p * q


def _out_body(p0_ref, p1_ref, xl_ref, Wout, bout, o_ref):
    D = Wout.shape[0]
    acc = p0_ref[0] + p1_ref[0] + xl_ref[...]
    o_ref[...] = _dot(acc, Wout[...]) * (1.0 / math.sqrt(D)) + bout[...]


def _full(shape):
    return pl.BlockSpec(shape, lambda i: (0,) * len(shape))


def _make_gather_kernel(D, H, E_pad):
    EPW = E_pad // _NW
    CH = 128
    SUBS = 1024 // CH
    OUTER = EPW // 1024
    SR = CH // 8
    mesh = plsc.VectorSubcoreMesh(core_axis_name="c", subcore_axis_name="s")

    @functools.partial(
        pl.kernel,
        out_type=[
            jax.ShapeDtypeStruct((E_pad // 8, D), jnp.float32),
            jax.ShapeDtypeStruct((E_pad // 8, D), jnp.float32),
        ],
        scratch_types=[
            pltpu.VMEM((SUBS, 128), jnp.int32),
            pltpu.VMEM((SUBS, 128), jnp.int32),
            pltpu.VMEM((CH, D), jnp.float32),
            pltpu.VMEM((CH, D), jnp.float32),
            pltpu.VMEM((CH, D), jnp.float32),
            pltpu.VMEM((CH, D), jnp.float32),
            pltpu.VMEM((SR, D), jnp.float32),
            pltpu.VMEM((SR, D), jnp.float32),
            pltpu.VMEM((SR, D), jnp.float32),
            pltpu.VMEM((SR, D), jnp.float32),
            pltpu.SemaphoreType.DMA,
            pltpu.SemaphoreType.DMA,
            pltpu.SemaphoreType.DMA,
            pltpu.SemaphoreType.DMA,
        ],
        mesh=mesh,
    )
    def g_kernel(P_hbm, dst2_hbm, src2_hbm, vL_hbm, vR_hbm,
                 idx_d, idx_s, gd0, gd1, gs0, gs1, sL0, sL1, sR0, sR1,
                 ls0, ls1, ws0, ws1):
        cid = lax.axis_index("c")
        sid = lax.axis_index("s")
        wid = sid * _NC + cid
        gds = (gd0, gd1)
        gss = (gs0, gs1)
        sLs = (sL0, sL1)
        sRs = (sR0, sR1)
        lsem = (ls0, ls1)
        wsem = (ws0, ws1)

        def outer(i, _):
            r0 = wid * (EPW // 128) + i * SUBS
            pltpu.sync_copy(dst2_hbm.at[pl.ds(r0, SUBS)], idx_d)
            pltpu.sync_copy(src2_hbm.at[pl.ds(r0, SUBS)], idx_s)
            v_base = wid * (EPW // 8) + i * (1024 // 8)
            pltpu.async_copy(P_hbm.at[idx_d.at[0]], gd0, ls0)
            pltpu.async_copy(P_hbm.at[idx_s.at[0]], gs0, ls0)

            def pair(tp, _2):
                for b in range(2):
                    t = 2 * tp + b
                    pltpu.make_async_copy(
                        P_hbm.at[idx_d.at[0]], gds[b], lsem[b]).wait()
                    pltpu.make_async_copy(
                        P_hbm.at[idx_s.at[0]], gss[b], lsem[b]).wait()
                    if b == 0:
                        pltpu.async_copy(
                            P_hbm.at[idx_d.at[t + 1]], gds[1], lsem[1])
                        pltpu.async_copy(
                            P_hbm.at[idx_s.at[t + 1]], gss[1], lsem[1])
                    else:
                        @pl.when(tp < SUBS // 2 - 1)
                        def _issue():
                            pltpu.async_copy(
                                P_hbm.at[idx_d.at[t + 1]], gds[0], lsem[0])
                            pltpu.async_copy(
                                P_hbm.at[idx_s.at[t + 1]], gss[0], lsem[0])
                    @pl.when(tp > 0)
                    def _drain():
                        pltpu.make_async_copy(
                            sLs[b], vL_hbm.at[pl.ds(v_base, SR)],
                            wsem[b]).wait()
                        pltpu.make_async_copy(
                            sRs[b], vR_hbm.at[pl.ds(v_base, SR)],
                            wsem[b]).wait()

                    gd_buf, gs_buf = gds[b], gss[b]
                    sumL, sumR = sLs[b], sRs[b]

                    def addrow(r, _3):
                        for j in range(8):
                            for k in range(H // 16):
                                val = (gd_buf[r * 8 + j, pl.ds(k * 16, 16)]
                                       + gs_buf[r * 8 + j,
                                                pl.ds(H + k * 16, 16)])
                                if j < 4:
                                    sumL[r, pl.ds(j * H + k * 16, 16)] = val
                                else:
                                    sumR[r, pl.ds((j - 4) * H + k * 16, 16)] = val
                        return _3
                    lax.fori_loop(0, SR, addrow, 0)

                    v0 = v_base + t * SR
                    pltpu.async_copy(sumL, vL_hbm.at[pl.ds(v0, SR)], wsem[b])
                    pltpu.async_copy(sumR, vR_hbm.at[pl.ds(v0, SR)], wsem[b])
                return _2

            lax.fori_loop(0, SUBS // 2, pair, 0)
            for b in range(2):
                pltpu.make_async_copy(
                    sLs[b], vL_hbm.at[pl.ds(v_base, SR)], wsem[b]).wait()
                pltpu.make_async_copy(
                    sRs[b], vR_hbm.at[pl.ds(v_base, SR)], wsem[b]).wait()
            return _

        lax.fori_loop(0, OUTER, outer, 0)

    return g_kernel


def _make_scatter_kernel(N, D, E_pad):
    EPW = E_pad // _NW
    CH = 64
    SUBS = 1024 // CH
    OUTER = EPW // 1024
    N_pad = -(-N // (_NS * 128)) * (_NS * 128)
    RPT = N_pad // _NS
    mesh = plsc.VectorSubcoreMesh(core_axis_name="c", subcore_axis_name="s")

    @functools.partial(
        pl.kernel,
        out_type=jax.ShapeDtypeStruct((_NC, N_pad, D), jnp.float32),
        scratch_types=[
            pltpu.VMEM((SUBS, CH), jnp.int32),
            pltpu.VMEM((SUBS, CH), jnp.int32),
            pltpu.VMEM((CH, D), jnp.float32),
            pltpu.VMEM((CH, D), jnp.float32),
            pltpu.VMEM((CH, D), jnp.float32),
            pltpu.VMEM((CH, D), jnp.float32),
            pltpu.VMEM_SHARED((N_pad, D), jnp.float32),
            pltpu.SemaphoreType.DMA,
            pltpu.SemaphoreType.DMA,
            pltpu.SemaphoreType.DMA,
            pltpu.SemaphoreType.DMA,
        ],
        mesh=mesh,
    )
    def s_kernel(xl_hbm, m_hbm, dst64_hbm, src64_hbm, part_hbm,
                 idx_d, idx_s, xl0, xl1, m0, m1, acc, ls0, ls1, ss0, ss1):
        cid = lax.axis_index("c")
        sid = lax.axis_index("s")
        wid = sid * _NC + cid
        xls = (xl0, xl1)
        ms = (m0, m1)
        lsem = (ls0, ls1)
        ssem = (ss0, ss1)

        def zrow(r, _):
            for k in range(D // 16):
                m0[r, pl.ds(k * 16, 16)] = jnp.zeros((16,), jnp.float32)
            return _
        lax.fori_loop(0, CH, zrow, 0)
        for j in range(RPT // CH):
            pltpu.sync_copy(m0, acc.at[pl.ds(sid * RPT + j * CH, CH)])
        plsc.subcore_barrier()

        def outer(i, _):
            r0 = wid * (EPW // CH) + i * SUBS
            pltpu.sync_copy(dst64_hbm.at[pl.ds(r0, SUBS)], idx_d)
            pltpu.sync_copy(src64_hbm.at[pl.ds(r0, SUBS)], idx_s)
            e_base = wid * EPW + i * 1024
            pltpu.async_copy(m_hbm.at[pl.ds(e_base, CH)], m0, ls0)
            pltpu.async_copy(xl_hbm.at[idx_s.at[0]], xl0, ls0)

            def pair(tp, _2):
                for b in range(2):
                    t = 2 * tp + b
                    pltpu.make_async_copy(
                        m_hbm.at[pl.ds(e_base, CH)], ms[b], lsem[b]).wait()
                    pltpu.make_async_copy(
                        xl_hbm.at[idx_s.at[0]], xls[b], lsem[b]).wait()
                    if b == 0:
                        @pl.when(tp > 0)
                        def _drain0():
                            pltpu.make_async_copy(
                                ms[1], acc.at[idx_d.at[0]], ssem[1]).wait()
                        e1 = e_base + (t + 1) * CH
                        pltpu.async_copy(
                            m_hbm.at[pl.ds(e1, CH)], ms[1], lsem[1])
                        pltpu.async_copy(
                            xl_hbm.at[idx_s.at[t + 1]], xls[1], lsem[1])
                    else:
                        @pl.when(tp < SUBS // 2 - 1)
                        def _drain_issue1():
                            pltpu.make_async_copy(
                                ms[0], acc.at[idx_d.at[0]], ssem[0]).wait()
                            e1 = e_base + (t + 1) * CH
                            pltpu.async_copy(
                                m_hbm.at[pl.ds(e1, CH)], ms[0], lsem[0])
                            pltpu.async_copy(
                                xl_hbm.at[idx_s.at[t + 1]], xls[0], lsem[0])

                    m_buf, xl_buf = ms[b], xls[b]

                    def mrow(r, _3):
                        for k in range(D // 16):
                            sl = pl.ds(k * 16, 16)
                            m_buf[r, sl] = m_buf[r, sl] * xl_buf[r, sl]
                        return _3
                    lax.fori_loop(0, CH, mrow, 0)

                    pltpu.async_copy(
                        ms[b], acc.at[idx_d.at[t]], ssem[b], add=True)
                return _2

            lax.fori_loop(0, SUBS // 2, pair, 0)
            for b in range(2):
                pltpu.make_async_copy(
                    ms[b], acc.at[idx_d.at[0]], ssem[b]).wait()
            return _

        lax.fori_loop(0, OUTER, outer, 0)
        plsc.subcore_barrier()
        pltpu.sync_copy(acc.at[pl.ds(sid * RPT, RPT)],
                        part_hbm.at[cid, pl.ds(sid * RPT, RPT)])

    return s_kernel, N_pad


def kernel(x, edge_index, edge_sh, edge_attr, W_pre, b_pre, W_node, b_node,
           G1, g1b, G2, g2b, fc1, fc2, l01, l02, W_out, b_out):
    N, D = x.shape
    E = edge_index.shape[1]
    EA = edge_attr.shape[1]
    H = fc2.shape[0]
    l01t = l01[:D]
    l01b = l01[D:]
    b_pre2 = b_pre.reshape(1, D)
    b_node2 = b_node.reshape(1, D)
    g1b2 = g1b.reshape(1, D)
    g2b2 = g2b.reshape(1, D)
    b_out2 = b_out.reshape(1, D)

    BN = 2000
    n_blocks = N // BN
    P, xl = pl.pallas_call(
        _node_body,
        grid=(n_blocks,),
        in_specs=[
            pl.BlockSpec((BN, D), lambda i: (i, 0)),
            _full((D, D)), _full((1, D)),
            _full((D, D)), _full((1, D)),
            _full((D, D)), _full((1, D)),
            _full((D, D)), _full((1, D)),
            _full((D, H)), _full((D, H)),
        ],
        out_specs=[
            pl.BlockSpec((BN, D), lambda i: (i, 0)),
            pl.BlockSpec((BN, D), lambda i: (i, 0)),
        ],
        out_shape=[
            jax.ShapeDtypeStruct((N, D), jnp.float32),
            jax.ShapeDtypeStruct((N, D), jnp.float32),
        ],
    )(x, W_pre, b_pre2, G1, g1b2, G2, g2b2, W_node, b_node2, l01t, l01b)

    EPW = -(-E // (_NW * 1024)) * 1024
    E_pad = EPW * _NW
    pad = E_pad - E
    dst = jnp.pad(edge_index[0], (0, pad))
    src = jnp.pad(edge_index[1], (0, pad))
    ea_pad = jnp.pad(edge_attr, ((0, pad), (0, 0)))
    sh_pad = jnp.pad(edge_sh, ((0, pad), (0, 0)))
    dst2 = dst.reshape(E_pad // 128, 128)
    src2 = src.reshape(E_pad // 128, 128)
    dst64 = dst.reshape(E_pad // 64, 64)
    src64 = src.reshape(E_pad // 64, 64)
    ea8 = ea_pad.reshape(E_pad // 8, 8 * EA)
    sh8 = sh_pad.reshape(E_pad // 8, 8)

    vL, vR = _make_gather_kernel(D, H, E_pad)(P, dst2, src2)

    BE = 8192
    B8 = BE // 8
    e_blocks = E_pad // BE
    m3 = pl.pallas_call(
        _edge_body,
        grid=(e_blocks,),
        in_specs=[
            pl.BlockSpec((B8, 8 * EA), lambda i: (i, 0)),
            pl.BlockSpec((B8, D), lambda i: (i, 0)),
            pl.BlockSpec((B8, D), lambda i: (i, 0)),
            pl.BlockSpec((B8, 8), lambda i: (i, 0)),
            _full((EA, H)), _full((H, D)), _full((H, D)),
        ],
        out_specs=pl.BlockSpec((B8, 8, D), lambda i: (i, 0, 0)),
        out_shape=jax.ShapeDtypeStruct((E_pad // 8, 8, D), jnp.float32),
    )(ea8, vL, vR, sh8, fc1, fc2, l02)
    m = m3.reshape(E_pad, D)

    s_kernel, N_pad = _make_scatter_kernel(N, D, E_pad)
    part = s_kernel(xl, m, dst64, src64)

    out = pl.pallas_call(
        _out_body,
        grid=(n_blocks,),
        in_specs=[
            pl.BlockSpec((1, BN, D), lambda i: (0, i, 0)),
            pl.BlockSpec((1, BN, D), lambda i: (1, i, 0)),
            pl.BlockSpec((BN, D), lambda i: (i, 0)),
            _full((D, D)), _full((1, D)),
        ],
        out_specs=pl.BlockSpec((BN, D), lambda i: (i, 0)),
        out_shape=jax.ShapeDtypeStruct((N, D), jnp.float32),
    )(part, part, xl, W_out, b_out2)
    return out


# speedup vs baseline: 2.6394x; 1.5305x over previous
import functools
import math

import jax
import jax.numpy as jnp
import numpy as np
from jax import lax
from jax.experimental import pallas as pl
from jax.experimental.pallas import tpu as pltpu
from jax.experimental.pallas import tpu_sc as plsc

_z = np.random.RandomState(0).randn(1000000)
_SSP_C = float(1.0 / np.sqrt(np.mean((np.logaddexp(0.0, _z) - np.log(2.0)) ** 2)))
_LOG2 = float(np.log(2.0))

_NC = 2
_NS = 16
_NW = _NC * _NS


def _ssp(x):
    return (jax.nn.softplus(x) - _LOG2) * _SSP_C


def _dot(a, b):
    return jax.lax.dot_general(a, b, (((1,), (0,)), ((), ())),
                               preferred_element_type=jnp.float32)


def _node_body(x_ref, Wpre, bpre, G1, g1b, G2, g2b, Wnode, bnode, l01t, l01b,
               a_ref, b_ref, xl_ref):
    x = x_ref[...]
    BN, D = x.shape
    H = l01t.shape[1]
    pre = _dot(x, Wpre[...]) * (1.0 / math.sqrt(D)) + bpre[...]
    s = 1.0 / math.sqrt(2 * D)
    a_ref[...] = _dot(pre, l01t[...]) * s
    b_ref[...] = _dot(pre, l01b[...]) * s
    h = jax.nn.silu(_dot(x, G1[...]) + g1b[...])
    xg = _dot(h, G2[...]) + g2b[...]
    xl_ref[...] = _dot(xg, Wnode[...]) * (1.0 / math.sqrt(D)) + bnode[...]


def _edge_body(ea_ref, vL_ref, vR_ref, sh_ref, fc1, fc2, l02, m_ref):
    H = fc2.shape[0]
    EA = fc1.shape[0]
    sf = 1.0 / math.sqrt(H)
    ea = ea_ref[...]
    vL = vL_ref[...]
    vR = vR_ref[...]
    sh = sh_ref[...]
    for g in range(8):
        ea_g = ea[:, g * EA:(g + 1) * EA]
        t = _ssp(_dot(ea_g, fc1[...]) * (1.0 / math.sqrt(EA)))
        p = _dot(t, fc2[...]) * sf
        v = vL if g < 4 else vR
        u = v[:, (g % 4) * H:((g % 4) + 1) * H]
        q = _dot(_ssp(u), l02[...]) * sf
        m_ref[:, g, :] = sh[:, g:g + 1] * p * q


def _out_body(p0_ref, p1_ref, xl_ref, Wout, bout, o_ref):
    D = Wout.shape[0]
    acc = p0_ref[0] + p1_ref[0] + xl_ref[...]
    o_ref[...] = _dot(acc, Wout[...]) * (1.0 / math.sqrt(D)) + bout[...]


def _full(shape):
    return pl.BlockSpec(shape, lambda i: (0,) * len(shape))


def _make_gather_kernel(D, H, E_pad):
    EPW = E_pad // _NW
    CH = 128
    SUBS = 1024 // CH
    OUTER = EPW // 1024
    SR = CH // 8
    mesh = plsc.VectorSubcoreMesh(core_axis_name="c", subcore_axis_name="s")

    @functools.partial(
        pl.kernel,
        out_type=[
            jax.ShapeDtypeStruct((E_pad // 8, D), jnp.float32),
            jax.ShapeDtypeStruct((E_pad // 8, D), jnp.float32),
        ],
        scratch_types=[
            pltpu.VMEM((SUBS, 128), jnp.int32),
            pltpu.VMEM((SUBS, 128), jnp.int32),
            pltpu.VMEM((CH, H), jnp.float32),
            pltpu.VMEM((CH, H), jnp.float32),
            pltpu.VMEM((CH, H), jnp.float32),
            pltpu.VMEM((CH, H), jnp.float32),
            pltpu.VMEM((SR, D), jnp.float32),
            pltpu.VMEM((SR, D), jnp.float32),
            pltpu.VMEM((SR, D), jnp.float32),
            pltpu.VMEM((SR, D), jnp.float32),
            pltpu.SemaphoreType.DMA,
            pltpu.SemaphoreType.DMA,
            pltpu.SemaphoreType.DMA,
            pltpu.SemaphoreType.DMA,
        ],
        compiler_params=pltpu.CompilerParams(use_tc_tiling_on_sc=False),
        mesh=mesh,
    )
    def g_kernel(a_hbm, b_hbm, dst2_hbm, src2_hbm, vL_hbm, vR_hbm,
                 idx_d, idx_s, gd0, gd1, gs0, gs1, sL0, sL1, sR0, sR1,
                 ls0, ls1, ws0, ws1):
        cid = lax.axis_index("c")
        sid = lax.axis_index("s")
        wid = sid * _NC + cid
        gds = (gd0, gd1)
        gss = (gs0, gs1)
        sLs = (sL0, sL1)
        sRs = (sR0, sR1)
        lsem = (ls0, ls1)
        wsem = (ws0, ws1)

        def outer(i, _):
            r0 = wid * (EPW // 128) + i * SUBS
            pltpu.sync_copy(dst2_hbm.at[pl.ds(r0, SUBS)], idx_d)
            pltpu.sync_copy(src2_hbm.at[pl.ds(r0, SUBS)], idx_s)
            v_base = wid * (EPW // 8) + i * (1024 // 8)
            pltpu.async_copy(a_hbm.at[idx_d.at[0]], gd0, ls0)
            pltpu.async_copy(b_hbm.at[idx_s.at[0]], gs0, ls0)

            def pair(tp, _2):
                for b in range(2):
                    t = 2 * tp + b
                    pltpu.make_async_copy(
                        a_hbm.at[idx_d.at[0]], gds[b], lsem[b]).wait()
                    pltpu.make_async_copy(
                        b_hbm.at[idx_s.at[0]], gss[b], lsem[b]).wait()
                    if b == 0:
                        pltpu.async_copy(
                            a_hbm.at[idx_d.at[t + 1]], gds[1], lsem[1])
                        pltpu.async_copy(
                            b_hbm.at[idx_s.at[t + 1]], gss[1], lsem[1])
                    else:
                        @pl.when(tp < SUBS // 2 - 1)
                        def _issue():
                            pltpu.async_copy(
                                a_hbm.at[idx_d.at[t + 1]], gds[0], lsem[0])
                            pltpu.async_copy(
                                b_hbm.at[idx_s.at[t + 1]], gss[0], lsem[0])
                    @pl.when(tp > 0)
                    def _drain():
                        pltpu.make_async_copy(
                            sLs[b], vL_hbm.at[pl.ds(v_base, SR)],
                            wsem[b]).wait()
                        pltpu.make_async_copy(
                            sRs[b], vR_hbm.at[pl.ds(v_base, SR)],
                            wsem[b]).wait()

                    gd_buf, gs_buf = gds[b], gss[b]
                    sumL, sumR = sLs[b], sRs[b]

                    def addrow(r, _3):
                        for j in range(8):
                            for k in range(H // 16):
                                val = (gd_buf[r * 8 + j, pl.ds(k * 16, 16)]
                                       + gs_buf[r * 8 + j, pl.ds(k * 16, 16)])
                                if j < 4:
                                    sumL[r, pl.ds(j * H + k * 16, 16)] = val
                                else:
                                    sumR[r, pl.ds((j - 4) * H + k * 16, 16)] = val
                        return _3
                    lax.fori_loop(0, SR, addrow, 0)

                    v0 = v_base + t * SR
                    pltpu.async_copy(sumL, vL_hbm.at[pl.ds(v0, SR)], wsem[b])
                    pltpu.async_copy(sumR, vR_hbm.at[pl.ds(v0, SR)], wsem[b])
                return _2

            lax.fori_loop(0, SUBS // 2, pair, 0)
            for b in range(2):
                pltpu.make_async_copy(
                    sLs[b], vL_hbm.at[pl.ds(v_base, SR)], wsem[b]).wait()
                pltpu.make_async_copy(
                    sRs[b], vR_hbm.at[pl.ds(v_base, SR)], wsem[b]).wait()
            return _

        lax.fori_loop(0, OUTER, outer, 0)

    return g_kernel


def _make_scatter_kernel(N, D, E_pad):
    EPW = E_pad // _NW
    CH = 64
    SUBS = 1024 // CH
    OUTER = EPW // 1024
    N_pad = -(-N // (_NS * 128)) * (_NS * 128)
    RPT = N_pad // _NS
    mesh = plsc.VectorSubcoreMesh(core_axis_name="c", subcore_axis_name="s")

    @functools.partial(
        pl.kernel,
        out_type=jax.ShapeDtypeStruct((_NC, N_pad, D), jnp.float32),
        scratch_types=[
            pltpu.VMEM((SUBS, CH), jnp.int32),
            pltpu.VMEM((SUBS, CH), jnp.int32),
            pltpu.VMEM((CH, D), jnp.float32),
            pltpu.VMEM((CH, D), jnp.float32),
            pltpu.VMEM((CH, D), jnp.float32),
            pltpu.VMEM((CH, D), jnp.float32),
            pltpu.VMEM_SHARED((N_pad, D), jnp.float32),
            pltpu.SemaphoreType.DMA,
            pltpu.SemaphoreType.DMA,
            pltpu.SemaphoreType.DMA,
            pltpu.SemaphoreType.DMA,
        ],
        mesh=mesh,
    )
    def s_kernel(xl_hbm, m_hbm, dst64_hbm, src64_hbm, part_hbm,
                 idx_d, idx_s, xl0, xl1, m0, m1, acc, ls0, ls1, ss0, ss1):
        cid = lax.axis_index("c")
        sid = lax.axis_index("s")
        wid = sid * _NC + cid
        xls = (xl0, xl1)
        ms = (m0, m1)
        lsem = (ls0, ls1)
        ssem = (ss0, ss1)

        def zrow(r, _):
            for k in range(D // 16):
                m0[r, pl.ds(k * 16, 16)] = jnp.zeros((16,), jnp.float32)
            return _
        lax.fori_loop(0, CH, zrow, 0)
        for j in range(RPT // CH):
            pltpu.sync_copy(m0, acc.at[pl.ds(sid * RPT + j * CH, CH)])
        plsc.subcore_barrier()

        def outer(i, _):
            r0 = wid * (EPW // CH) + i * SUBS
            pltpu.sync_copy(dst64_hbm.at[pl.ds(r0, SUBS)], idx_d)
            pltpu.sync_copy(src64_hbm.at[pl.ds(r0, SUBS)], idx_s)
            e_base = wid * EPW + i * 1024
            pltpu.async_copy(m_hbm.at[pl.ds(e_base, CH)], m0, ls0)
            pltpu.async_copy(xl_hbm.at[idx_s.at[0]], xl0, ls0)

            def pair(tp, _2):
                for b in range(2):
                    t = 2 * tp + b
                    pltpu.make_async_copy(
                        m_hbm.at[pl.ds(e_base, CH)], ms[b], lsem[b]).wait()
                    pltpu.make_async_copy(
                        xl_hbm.at[idx_s.at[0]], xls[b], lsem[b]).wait()
                    if b == 0:
                        @pl.when(tp > 0)
                        def _drain0():
                            pltpu.make_async_copy(
                                ms[1], acc.at[idx_d.at[0]], ssem[1]).wait()
                        e1 = e_base + (t + 1) * CH
                        pltpu.async_copy(
                            m_hbm.at[pl.ds(e1, CH)], ms[1], lsem[1])
                        pltpu.async_copy(
                            xl_hbm.at[idx_s.at[t + 1]], xls[1], lsem[1])
                    else:
                        @pl.when(tp < SUBS // 2 - 1)
                        def _drain_issue1():
                            pltpu.make_async_copy(
                                ms[0], acc.at[idx_d.at[0]], ssem[0]).wait()
                            e1 = e_base + (t + 1) * CH
                            pltpu.async_copy(
                                m_hbm.at[pl.ds(e1, CH)], ms[0], lsem[0])
                            pltpu.async_copy(
                                xl_hbm.at[idx_s.at[t + 1]], xls[0], lsem[0])

                    m_buf, xl_buf = ms[b], xls[b]

                    def mrow(r, _3):
                        for k in range(D // 16):
                            sl = pl.ds(k * 16, 16)
                            m_buf[r, sl] = m_buf[r, sl] * xl_buf[r, sl]
                        return _3
                    lax.fori_loop(0, CH, mrow, 0)

                    pltpu.async_copy(
                        ms[b], acc.at[idx_d.at[t]], ssem[b], add=True)
                return _2

            lax.fori_loop(0, SUBS // 2, pair, 0)
            for b in range(2):
                pltpu.make_async_copy(
                    ms[b], acc.at[idx_d.at[0]], ssem[b]).wait()
            return _

        lax.fori_loop(0, OUTER, outer, 0)
        plsc.subcore_barrier()
        pltpu.sync_copy(acc.at[pl.ds(sid * RPT, RPT)],
                        part_hbm.at[cid, pl.ds(sid * RPT, RPT)])

    return s_kernel, N_pad


def kernel(x, edge_index, edge_sh, edge_attr, W_pre, b_pre, W_node, b_node,
           G1, g1b, G2, g2b, fc1, fc2, l01, l02, W_out, b_out):
    N, D = x.shape
    E = edge_index.shape[1]
    EA = edge_attr.shape[1]
    H = fc2.shape[0]
    l01t = l01[:D]
    l01b = l01[D:]
    b_pre2 = b_pre.reshape(1, D)
    b_node2 = b_node.reshape(1, D)
    g1b2 = g1b.reshape(1, D)
    g2b2 = g2b.reshape(1, D)
    b_out2 = b_out.reshape(1, D)

    BN = 2000
    n_blocks = N // BN
    a_nd, b_nd, xl = pl.pallas_call(
        _node_body,
        grid=(n_blocks,),
        in_specs=[
            pl.BlockSpec((BN, D), lambda i: (i, 0)),
            _full((D, D)), _full((1, D)),
            _full((D, D)), _full((1, D)),
            _full((D, D)), _full((1, D)),
            _full((D, D)), _full((1, D)),
            _full((D, H)), _full((D, H)),
        ],
        out_specs=[
            pl.BlockSpec((BN, H), lambda i: (i, 0)),
            pl.BlockSpec((BN, H), lambda i: (i, 0)),
            pl.BlockSpec((BN, D), lambda i: (i, 0)),
        ],
        out_shape=[
            jax.ShapeDtypeStruct((N, H), jnp.float32),
            jax.ShapeDtypeStruct((N, H), jnp.float32),
            jax.ShapeDtypeStruct((N, D), jnp.float32),
        ],
    )(x, W_pre, b_pre2, G1, g1b2, G2, g2b2, W_node, b_node2, l01t, l01b)

    EPW = -(-E // (_NW * 1024)) * 1024
    E_pad = EPW * _NW
    pad = E_pad - E
    dst = jnp.pad(edge_index[0], (0, pad))
    src = jnp.pad(edge_index[1], (0, pad))
    ea_pad = jnp.pad(edge_attr, ((0, pad), (0, 0)))
    sh_pad = jnp.pad(edge_sh, ((0, pad), (0, 0)))
    dst2 = dst.reshape(E_pad // 128, 128)
    src2 = src.reshape(E_pad // 128, 128)
    dst64 = dst.reshape(E_pad // 64, 64)
    src64 = src.reshape(E_pad // 64, 64)
    ea8 = ea_pad.reshape(E_pad // 8, 8 * EA)
    sh8 = sh_pad.reshape(E_pad // 8, 8)

    vL, vR = _make_gather_kernel(D, H, E_pad)(a_nd, b_nd, dst2, src2)

    BE = 8192
    B8 = BE // 8
    e_blocks = E_pad // BE
    m3 = pl.pallas_call(
        _edge_body,
        grid=(e_blocks,),
        in_specs=[
            pl.BlockSpec((B8, 8 * EA), lambda i: (i, 0)),
            pl.BlockSpec((B8, D), lambda i: (i, 0)),
            pl.BlockSpec((B8, D), lambda i: (i, 0)),
            pl.BlockSpec((B8, 8), lambda i: (i, 0)),
            _full((EA, H)), _full((H, D)), _full((H, D)),
        ],
        out_specs=pl.BlockSpec((B8, 8, D), lambda i: (i, 0, 0)),
        out_shape=jax.ShapeDtypeStruct((E_pad // 8, 8, D), jnp.float32),
    )(ea8, vL, vR, sh8, fc1, fc2, l02)
    m = m3.reshape(E_pad, D)

    s_kernel, N_pad = _make_scatter_kernel(N, D, E_pad)
    part = s_kernel(xl, m, dst64, src64)

    out = pl.pallas_call(
        _out_body,
        grid=(n_blocks,),
        in_specs=[
            pl.BlockSpec((1, BN, D), lambda i: (0, i, 0)),
            pl.BlockSpec((1, BN, D), lambda i: (1, i, 0)),
            pl.BlockSpec((BN, D), lambda i: (i, 0)),
            _full((D, D)), _full((1, D)),
        ],
        out_specs=pl.BlockSpec((BN, D), lambda i: (i, 0)),
        out_shape=jax.ShapeDtypeStruct((N, D), jnp.float32),
    )(part, part, xl, W_out, b_out2)
    return out


# speedup vs baseline: 3.1689x; 1.2006x over previous
import functools
import math

import jax
import jax.numpy as jnp
import numpy as np
from jax import lax
from jax.experimental import pallas as pl
from jax.experimental.pallas import tpu as pltpu
from jax.experimental.pallas import tpu_sc as plsc

_z = np.random.RandomState(0).randn(1000000)
_SSP_C = float(1.0 / np.sqrt(np.mean((np.logaddexp(0.0, _z) - np.log(2.0)) ** 2)))
_LOG2 = float(np.log(2.0))

_NC = 2
_NS = 16
_NW = _NC * _NS


def _ssp(x):
    return (jax.nn.softplus(x) - _LOG2) * _SSP_C


def _dot(a, b):
    return jax.lax.dot_general(a, b, (((1,), (0,)), ((), ())),
                               preferred_element_type=jnp.float32)


def _node_body(x_ref, Wpre, bpre, G1, g1b, G2, g2b, Wnode, bnode, l01t, l01b,
               a_ref, b_ref, xl_ref):
    x = x_ref[...]
    BN, D = x.shape
    H = l01t.shape[1]
    pre = _dot(x, Wpre[...]) * (1.0 / math.sqrt(D)) + bpre[...]
    s = 1.0 / math.sqrt(2 * D)
    a_ref[...] = _dot(pre, l01t[...]) * s
    b_ref[...] = _dot(pre, l01b[...]) * s
    h = jax.nn.silu(_dot(x, G1[...]) + g1b[...])
    xg = _dot(h, G2[...]) + g2b[...]
    xl_ref[...] = _dot(xg, Wnode[...]) * (1.0 / math.sqrt(D)) + bnode[...]


def _edge_body(ea_ref, vL_ref, vR_ref, sh_ref, fc1, fc2, l02, m_ref):
    H = fc2.shape[0]
    EA = fc1.shape[0]
    sf = 1.0 / math.sqrt(H)
    ea = ea_ref[...]
    vL = vL_ref[...]
    vR = vR_ref[...]
    sh = sh_ref[...]
    for g in range(8):
        ea_g = ea[:, g * EA:(g + 1) * EA]
        t = _ssp(_dot(ea_g, fc1[...]) * (1.0 / math.sqrt(EA)))
        t = t * sh[:, g:g + 1]
        p = _dot(t, fc2[...]) * sf
        v = vL if g < 4 else vR
        u = v[:, (g % 4) * H:((g % 4) + 1) * H]
        q = _dot(_ssp(u), l02[...]) * sf
        m_ref[:, g, :] = p * q


def _out_body(p0_ref, p1_ref, xl_ref, Wout, bout, o_ref):
    D = Wout.shape[0]
    acc = p0_ref[0] + p1_ref[0] + xl_ref[...]
    o_ref[...] = _dot(acc, Wout[...]) * (1.0 / math.sqrt(D)) + bout[...]


def _full(shape):
    return pl.BlockSpec(shape, lambda i: (0,) * len(shape))


def _make_gather_kernel(D, H, E_pad, seg, nseg):
    E_seg = E_pad // nseg
    SEG = seg * E_seg
    EPW = E_seg // _NW
    CH = 128
    SUBS = 1024 // CH
    OUTER = EPW // 1024
    SR = CH // 8
    mesh = plsc.VectorSubcoreMesh(core_axis_name="c", subcore_axis_name="s")

    @functools.partial(
        pl.kernel,
        out_type=[
            jax.ShapeDtypeStruct((E_seg // 8, D), jnp.float32),
            jax.ShapeDtypeStruct((E_seg // 8, D), jnp.float32),
        ],
        scratch_types=[
            pltpu.VMEM((SUBS, 128), jnp.int32),
            pltpu.VMEM((SUBS, 128), jnp.int32),
            pltpu.VMEM((CH, H), jnp.float32),
            pltpu.VMEM((CH, H), jnp.float32),
            pltpu.VMEM((CH, H), jnp.float32),
            pltpu.VMEM((CH, H), jnp.float32),
            pltpu.VMEM((SR, D), jnp.float32),
            pltpu.VMEM((SR, D), jnp.float32),
            pltpu.VMEM((SR, D), jnp.float32),
            pltpu.VMEM((SR, D), jnp.float32),
            pltpu.SemaphoreType.DMA,
            pltpu.SemaphoreType.DMA,
            pltpu.SemaphoreType.DMA,
            pltpu.SemaphoreType.DMA,
        ],
        compiler_params=pltpu.CompilerParams(use_tc_tiling_on_sc=False),
        mesh=mesh,
    )
    def g_kernel(a_hbm, b_hbm, dst2_hbm, src2_hbm, vL_hbm, vR_hbm,
                 idx_d, idx_s, gd0, gd1, gs0, gs1, sL0, sL1, sR0, sR1,
                 ls0, ls1, ws0, ws1):
        cid = lax.axis_index("c")
        sid = lax.axis_index("s")
        wid = sid * _NC + cid
        gds = (gd0, gd1)
        gss = (gs0, gs1)
        sLs = (sL0, sL1)
        sRs = (sR0, sR1)
        lsem = (ls0, ls1)
        wsem = (ws0, ws1)

        def outer(i, _):
            r0 = SEG // 128 + wid * (EPW // 128) + i * SUBS
            pltpu.sync_copy(dst2_hbm.at[pl.ds(r0, SUBS)], idx_d)
            pltpu.sync_copy(src2_hbm.at[pl.ds(r0, SUBS)], idx_s)
            v_base = wid * (EPW // 8) + i * (1024 // 8)
            pltpu.async_copy(a_hbm.at[idx_d.at[0]], gd0, ls0)
            pltpu.async_copy(b_hbm.at[idx_s.at[0]], gs0, ls0)

            def pair(tp, _2):
                for b in range(2):
                    t = 2 * tp + b
                    pltpu.make_async_copy(
                        a_hbm.at[idx_d.at[0]], gds[b], lsem[b]).wait()
                    pltpu.make_async_copy(
                        b_hbm.at[idx_s.at[0]], gss[b], lsem[b]).wait()
                    if b == 0:
                        pltpu.async_copy(
                            a_hbm.at[idx_d.at[t + 1]], gds[1], lsem[1])
                        pltpu.async_copy(
                            b_hbm.at[idx_s.at[t + 1]], gss[1], lsem[1])
                    else:
                        @pl.when(tp < SUBS // 2 - 1)
                        def _issue():
                            pltpu.async_copy(
                                a_hbm.at[idx_d.at[t + 1]], gds[0], lsem[0])
                            pltpu.async_copy(
                                b_hbm.at[idx_s.at[t + 1]], gss[0], lsem[0])
                    @pl.when(tp > 0)
                    def _drain():
                        pltpu.make_async_copy(
                            sLs[b], vL_hbm.at[pl.ds(v_base, SR)],
                            wsem[b]).wait()
                        pltpu.make_async_copy(
                            sRs[b], vR_hbm.at[pl.ds(v_base, SR)],
                            wsem[b]).wait()

                    gd_buf, gs_buf = gds[b], gss[b]
                    sumL, sumR = sLs[b], sRs[b]

                    def addrow(r, _3):
                        for j in range(8):
                            for k in range(H // 16):
                                val = (gd_buf[r * 8 + j, pl.ds(k * 16, 16)]
                                       + gs_buf[r * 8 + j, pl.ds(k * 16, 16)])
                                if j < 4:
                                    sumL[r, pl.ds(j * H + k * 16, 16)] = val
                                else:
                                    sumR[r, pl.ds((j - 4) * H + k * 16, 16)] = val
                        return _3
                    lax.fori_loop(0, SR, addrow, 0)

                    v0 = v_base + t * SR
                    pltpu.async_copy(sumL, vL_hbm.at[pl.ds(v0, SR)], wsem[b])
                    pltpu.async_copy(sumR, vR_hbm.at[pl.ds(v0, SR)], wsem[b])
                return _2

            lax.fori_loop(0, SUBS // 2, pair, 0)
            for b in range(2):
                pltpu.make_async_copy(
                    sLs[b], vL_hbm.at[pl.ds(v_base, SR)], wsem[b]).wait()
                pltpu.make_async_copy(
                    sRs[b], vR_hbm.at[pl.ds(v_base, SR)], wsem[b]).wait()
            return _

        lax.fori_loop(0, OUTER, outer, 0)

    return g_kernel


def _make_scatter_kernel(N, D, E_pad, seg, nseg):
    E_seg = E_pad // nseg
    SEG = seg * E_seg
    EPW = E_seg // _NW
    CH = 64
    SUBS = 1024 // CH
    OUTER = EPW // 1024
    N_pad = -(-N // (_NS * 128)) * (_NS * 128)
    RPT = N_pad // _NS
    mesh = plsc.VectorSubcoreMesh(core_axis_name="c", subcore_axis_name="s")

    def body(xl_hbm, m_hbm, dst64_hbm, src64_hbm, *rest):
        if seg > 0:
            (pin_hbm, part_hbm,
             idx_d, idx_s, xl0, xl1, m0, m1, acc, ls0, ls1, ss0, ss1) = rest
        else:
            (part_hbm,
             idx_d, idx_s, xl0, xl1, m0, m1, acc, ls0, ls1, ss0, ss1) = rest
        cid = lax.axis_index("c")
        sid = lax.axis_index("s")
        wid = sid * _NC + cid
        xls = (xl0, xl1)
        ms = (m0, m1)
        lsem = (ls0, ls1)
        ssem = (ss0, ss1)

        if seg > 0:
            pltpu.sync_copy(pin_hbm.at[cid, pl.ds(sid * RPT, RPT)],
                            acc.at[pl.ds(sid * RPT, RPT)])
        else:
            def zrow(r, _):
                for k in range(D // 16):
                    m0[r, pl.ds(k * 16, 16)] = jnp.zeros((16,), jnp.float32)
                return _
            lax.fori_loop(0, CH, zrow, 0)
            for j in range(RPT // CH):
                pltpu.sync_copy(m0, acc.at[pl.ds(sid * RPT + j * CH, CH)])
        plsc.subcore_barrier()

        def outer(i, _):
            r0 = SEG // CH + wid * (EPW // CH) + i * SUBS
            pltpu.sync_copy(dst64_hbm.at[pl.ds(r0, SUBS)], idx_d)
            pltpu.sync_copy(src64_hbm.at[pl.ds(r0, SUBS)], idx_s)
            e_base = wid * EPW + i * 1024
            pltpu.async_copy(m_hbm.at[pl.ds(e_base, CH)], m0, ls0)
            pltpu.async_copy(xl_hbm.at[idx_s.at[0]], xl0, ls0)

            def pair(tp, _2):
                for b in range(2):
                    t = 2 * tp + b
                    pltpu.make_async_copy(
                        m_hbm.at[pl.ds(e_base, CH)], ms[b], lsem[b]).wait()
                    pltpu.make_async_copy(
                        xl_hbm.at[idx_s.at[0]], xls[b], lsem[b]).wait()
                    if b == 0:
                        @pl.when(tp > 0)
                        def _drain0():
                            pltpu.make_async_copy(
                                ms[1], acc.at[idx_d.at[0]], ssem[1]).wait()
                        e1 = e_base + (t + 1) * CH
                        pltpu.async_copy(
                            m_hbm.at[pl.ds(e1, CH)], ms[1], lsem[1])
                        pltpu.async_copy(
                            xl_hbm.at[idx_s.at[t + 1]], xls[1], lsem[1])
                    else:
                        @pl.when(tp < SUBS // 2 - 1)
                        def _drain_issue1():
                            pltpu.make_async_copy(
                                ms[0], acc.at[idx_d.at[0]], ssem[0]).wait()
                            e1 = e_base + (t + 1) * CH
                            pltpu.async_copy(
                                m_hbm.at[pl.ds(e1, CH)], ms[0], lsem[0])
                            pltpu.async_copy(
                                xl_hbm.at[idx_s.at[t + 1]], xls[0], lsem[0])

                    m_buf, xl_buf = ms[b], xls[b]

                    def mrow(r, _3):
                        for k in range(D // 16):
                            sl = pl.ds(k * 16, 16)
                            m_buf[r, sl] = m_buf[r, sl] * xl_buf[r, sl]
                        return _3
                    lax.fori_loop(0, CH, mrow, 0)

                    pltpu.async_copy(
                        ms[b], acc.at[idx_d.at[t]], ssem[b], add=True)
                return _2

            lax.fori_loop(0, SUBS // 2, pair, 0)
            for b in range(2):
                pltpu.make_async_copy(
                    ms[b], acc.at[idx_d.at[0]], ssem[b]).wait()
            return _

        lax.fori_loop(0, OUTER, outer, 0)
        plsc.subcore_barrier()
        pltpu.sync_copy(acc.at[pl.ds(sid * RPT, RPT)],
                        part_hbm.at[cid, pl.ds(sid * RPT, RPT)])

    s_kernel = pl.kernel(
        body,
        out_type=jax.ShapeDtypeStruct((_NC, N_pad, D), jnp.float32),
        scratch_types=[
            pltpu.VMEM((SUBS, CH), jnp.int32),
            pltpu.VMEM((SUBS, CH), jnp.int32),
            pltpu.VMEM((CH, D), jnp.float32),
            pltpu.VMEM((CH, D), jnp.float32),
            pltpu.VMEM((CH, D), jnp.float32),
            pltpu.VMEM((CH, D), jnp.float32),
            pltpu.VMEM_SHARED((N_pad, D), jnp.float32),
            pltpu.SemaphoreType.DMA,
            pltpu.SemaphoreType.DMA,
            pltpu.SemaphoreType.DMA,
            pltpu.SemaphoreType.DMA,
        ],
        mesh=mesh,
    )
    return s_kernel, N_pad


def kernel(x, edge_index, edge_sh, edge_attr, W_pre, b_pre, W_node, b_node,
           G1, g1b, G2, g2b, fc1, fc2, l01, l02, W_out, b_out):
    N, D = x.shape
    E = edge_index.shape[1]
    EA = edge_attr.shape[1]
    H = fc2.shape[0]
    l01t = l01[:D]
    l01b = l01[D:]
    b_pre2 = b_pre.reshape(1, D)
    b_node2 = b_node.reshape(1, D)
    g1b2 = g1b.reshape(1, D)
    g2b2 = g2b.reshape(1, D)
    b_out2 = b_out.reshape(1, D)

    BN = 2000
    n_blocks = N // BN
    a_nd, b_nd, xl = pl.pallas_call(
        _node_body,
        grid=(n_blocks,),
        in_specs=[
            pl.BlockSpec((BN, D), lambda i: (i, 0)),
            _full((D, D)), _full((1, D)),
            _full((D, D)), _full((1, D)),
            _full((D, D)), _full((1, D)),
            _full((D, D)), _full((1, D)),
            _full((D, H)), _full((D, H)),
        ],
        out_specs=[
            pl.BlockSpec((BN, H), lambda i: (i, 0)),
            pl.BlockSpec((BN, H), lambda i: (i, 0)),
            pl.BlockSpec((BN, D), lambda i: (i, 0)),
        ],
        out_shape=[
            jax.ShapeDtypeStruct((N, H), jnp.float32),
            jax.ShapeDtypeStruct((N, H), jnp.float32),
            jax.ShapeDtypeStruct((N, D), jnp.float32),
        ],
    )(x, W_pre, b_pre2, G1, g1b2, G2, g2b2, W_node, b_node2, l01t, l01b)

    EPW = -(-E // (_NW * 1024)) * 1024
    E_pad = EPW * _NW
    pad = E_pad - E
    dst = jnp.pad(edge_index[0], (0, pad))
    src = jnp.pad(edge_index[1], (0, pad))
    ea_pad = jnp.pad(edge_attr, ((0, pad), (0, 0)))
    sh_pad = jnp.pad(edge_sh, ((0, pad), (0, 0)))
    dst2 = dst.reshape(E_pad // 128, 128)
    src2 = src.reshape(E_pad // 128, 128)
    dst64 = dst.reshape(E_pad // 64, 64)
    src64 = src.reshape(E_pad // 64, 64)
    ea8 = ea_pad.reshape(E_pad // 8, 8 * EA)
    sh8 = sh_pad.reshape(E_pad // 8, 8)

    NSEG = 2
    E_seg = E_pad // NSEG
    BE = 8192
    B8 = BE // 8
    seg_blocks = E_seg // BE
    part = None
    for seg in range(NSEG):
        vL, vR = _make_gather_kernel(D, H, E_pad, seg, NSEG)(
            a_nd, b_nd, dst2, src2)
        off = seg * seg_blocks
        m3 = pl.pallas_call(
            _edge_body,
            grid=(seg_blocks,),
            in_specs=[
                pl.BlockSpec((B8, 8 * EA), lambda i, o=off: (i + o, 0)),
                pl.BlockSpec((B8, D), lambda i: (i, 0)),
                pl.BlockSpec((B8, D), lambda i: (i, 0)),
                pl.BlockSpec((B8, 8), lambda i, o=off: (i + o, 0)),
                _full((EA, H)), _full((H, D)), _full((H, D)),
            ],
            out_specs=pl.BlockSpec((B8, 8, D), lambda i: (i, 0, 0)),
            out_shape=jax.ShapeDtypeStruct((E_seg // 8, 8, D), jnp.float32),
        )(ea8, vL, vR, sh8, fc1, fc2, l02)
        m = m3.reshape(E_seg, D)
        s_kernel, N_pad = _make_scatter_kernel(N, D, E_pad, seg, NSEG)
        if seg == 0:
            part = s_kernel(xl, m, dst64, src64)
        else:
            part = s_kernel(xl, m, dst64, src64, part)

    out = pl.pallas_call(
        _out_body,
        grid=(n_blocks,),
        in_specs=[
            pl.BlockSpec((1, BN, D), lambda i: (0, i, 0)),
            pl.BlockSpec((1, BN, D), lambda i: (1, i, 0)),
            pl.BlockSpec((BN, D), lambda i: (i, 0)),
            _full((D, D)), _full((1, D)),
        ],
        out_specs=pl.BlockSpec((BN, D), lambda i: (i, 0)),
        out_shape=jax.ShapeDtypeStruct((N, D), jnp.float32),
    )(part, part, xl, W_out, b_out2)
    return out
